# baseline (speedup 1.0000x reference)
import functools
import math

import jax
import jax.numpy as jnp
from jax import lax
from jax.experimental import pallas as pl
from jax.experimental.pallas import tpu as pltpu
from jax.experimental.pallas import tpu_sc as plsc

N_TOK = 4096
C_DIM = 1024
E_NUM = 8
D_DIM = 3072
BM = 512
G_BLOCKS = 23
XS_ROWS = (G_BLOCKS + 1) * BM
NW = 32
A_PER_W = 2 * N_TOK // NW
T_PER_W = N_TOK // NW


def _router_body(x_ref, wr_ref, idxT_ref, wT_ref):
    xb = x_ref[...]
    wr = wr_ref[...]
    logits = lax.dot_general(
        xb, wr, (((1,), (1,)), ((), ())), preferred_element_type=jnp.float32
    )
    bt, e = logits.shape
    iota_e = lax.broadcasted_iota(jnp.int32, (bt, e), 1)
    i1 = jnp.argmax(logits, axis=1)
    m1 = jnp.max(logits, axis=1)
    masked = jnp.where(iota_e == i1[:, None], -jnp.inf, logits)
    i2 = jnp.argmax(masked, axis=1)
    m2 = jnp.max(masked, axis=1)
    w0 = 1.0 / (1.0 + jnp.exp(m2 - m1))
    w1 = 1.0 - w0
    idxT_ref[...] = jnp.concatenate(
        [i1.astype(jnp.int32)[None, :], i2.astype(jnp.int32)[None, :]], axis=0
    )
    wT_ref[...] = jnp.concatenate([w0[None, :], w1[None, :]], axis=0)


def _router(x_flat, Wr):
    nb = N_TOK // BM
    return pl.pallas_call(
        _router_body,
        grid=(nb,),
        in_specs=[
            pl.BlockSpec((BM, C_DIM), lambda tb: (tb, 0)),
            pl.BlockSpec((E_NUM, C_DIM), lambda tb: (0, 0)),
        ],
        out_specs=[
            pl.BlockSpec((2, BM), lambda tb: (0, tb)),
            pl.BlockSpec((2, BM), lambda tb: (0, tb)),
        ],
        out_shape=[
            jax.ShapeDtypeStruct((2, N_TOK), jnp.int32),
            jax.ShapeDtypeStruct((2, N_TOK), jnp.float32),
        ],
    )(x_flat, Wr)


def _splat(vec16, e, lane):
    s = lax.reduce_sum_p.bind(
        jnp.where(lane == e, vec16, 0), axes=(0,)
    )
    return jnp.broadcast_to(s, (16,))


def _dispatch_body(eflat, x_hbm, pos_hbm, xs_hbm, ee_hbm, xsid_hbm, outid_hbm,
                   ev_all, posflat, pos2d, xbuf, m_ee, m_xs, m_out, sem):
    wid = lax.axis_index("s") * 2 + lax.axis_index("c")
    lane = lax.iota(jnp.int32, 16)
    pltpu.sync_copy(eflat, ev_all)

    start_chunk = wid * (A_PER_W // 16)

    def scan_body(i, carry):
        hist, pre = carry
        pre = jnp.where(jnp.broadcast_to(i == start_chunk, (16,)), hist, pre)
        v = ev_all[pl.ds(i * 16, 16)]
        for e in range(E_NUM):
            pc = plsc.all_reduce_population_count(v == e)
            hist = jnp.where(lane == e, hist + pc, hist)
        return hist, pre

    zeros16 = jnp.zeros((16,), jnp.int32)
    hist, pre = lax.fori_loop(0, (2 * N_TOK) // 16, scan_body, (zeros16, zeros16))

    nb = (hist + (BM - 1)) >> 9
    nb = jnp.where(lane < E_NUM, nb, 0)
    blk_incl = plsc.cumsum(nb)
    blk_off = blk_incl - nb
    base_lane = blk_off * BM + pre

    bases = [_splat(base_lane, e, lane) for e in range(E_NUM)]
    ends = [_splat(blk_incl, e, lane) for e in range(E_NUM)]
    a_tot = ends[E_NUM - 1]

    for ci in range(2):
        bvec = lane + ci * 16
        eob = jnp.zeros((16,), jnp.int32)
        for e in range(E_NUM):
            eob = eob + jnp.where(bvec >= ends[e], 1, 0)
        act = bvec < a_tot
        m_ee[pl.ds(ci * 16, 16)] = jnp.minimum(eob, E_NUM - 1)
        m_xs[pl.ds(ci * 16, 16)] = jnp.where(act, bvec, a_tot - 1)
        m_out[pl.ds(ci * 16, 16)] = jnp.where(
            act, bvec, jnp.broadcast_to(G_BLOCKS, (16,))
        )

    @pl.when(wid == 0)
    def _write_meta():
        pltpu.sync_copy(m_ee, ee_hbm)
        pltpu.sync_copy(m_xs, xsid_hbm)
        pltpu.sync_copy(m_out, outid_hbm)

    for i in range(A_PER_W // 16):
        v = ev_all[pl.ds((start_chunk + i) * 16, 16)]
        pos = jnp.zeros((16,), jnp.int32)
        for e in range(E_NUM):
            m = v == e
            cs = plsc.cumsum(m.astype(jnp.int32))
            pos = jnp.where(m, bases[e] + cs - 1, pos)
            bases[e] = bases[e] + plsc.all_reduce_population_count(m)
        posflat[pl.ds(i * 16, 16)] = pos
        pos2d[i // 2, pl.ds((i % 2) * 16, 16)] = pos

    pltpu.sync_copy(posflat, pos_hbm.at[pl.ds(wid * A_PER_W, A_PER_W)])

    rowbase = (wid % 16) * A_PER_W
    for j in range(A_PER_W // 32):
        pltpu.sync_copy(x_hbm.at[pl.ds(rowbase + j * 32, 32)], xbuf)
        pltpu.async_copy(xbuf, xs_hbm.at[pos2d.at[j]], sem).wait()


def _dispatch(eflat, x_flat):
    mesh = plsc.VectorSubcoreMesh(core_axis_name="c", subcore_axis_name="s")
    f = functools.partial(
        pl.kernel,
        out_type=[
            jax.ShapeDtypeStruct((2 * N_TOK,), jnp.int32),
            jax.ShapeDtypeStruct((XS_ROWS, C_DIM), jnp.float32),
            jax.ShapeDtypeStruct((32,), jnp.int32),
            jax.ShapeDtypeStruct((32,), jnp.int32),
            jax.ShapeDtypeStruct((32,), jnp.int32),
        ],
        mesh=mesh,
        compiler_params=pltpu.CompilerParams(needs_layout_passes=False),
        scratch_types=[
            pltpu.VMEM((2 * N_TOK,), jnp.int32),
            pltpu.VMEM((A_PER_W,), jnp.int32),
            pltpu.VMEM((A_PER_W // 32, 32), jnp.int32),
            pltpu.VMEM((32, C_DIM), jnp.float32),
            pltpu.VMEM((32,), jnp.int32),
            pltpu.VMEM((32,), jnp.int32),
            pltpu.VMEM((32,), jnp.int32),
            pltpu.SemaphoreType.DMA,
        ],
    )(_dispatch_body)
    return f(eflat, x_flat)


def _ffn_body(ee_ref, xsid_ref, outid_ref, xs_ref, w1_ref, b1_ref, w2_ref,
              b2_ref, out_ref):
    g = pl.program_id(0)
    dc = pl.program_id(1)
    active = outid_ref[g] != G_BLOCKS

    @pl.when(active)
    def _():
        xb = xs_ref[...].astype(jnp.bfloat16)
        w1 = w1_ref[0]
        h = lax.dot_general(
            xb, w1, (((1,), (1,)), ((), ())), preferred_element_type=jnp.float32
        ) + b1_ref[0]
        h = 0.5 * h * (1.0 + lax.erf(h * (1.0 / math.sqrt(2.0))))
        w2 = w2_ref[0]
        o = lax.dot_general(
            h.astype(jnp.bfloat16), w2, (((1,), (1,)), ((), ())),
            preferred_element_type=jnp.float32,
        )

        @pl.when(dc == 0)
        def _init():
            out_ref[...] = o + b2_ref[0]

        @pl.when(dc > 0)
        def _acc():
            out_ref[...] = out_ref[...] + o


def _ffn(ee, xsid, outid, xs, W1, b1, W2, b2):
    dcb = 1024
    ndc = D_DIM // dcb
    b1r = b1[:, None, :]
    b2r = b2[:, None, :]
    grid_spec = pltpu.PrefetchScalarGridSpec(
        num_scalar_prefetch=3,
        grid=(G_BLOCKS, ndc),
        in_specs=[
            pl.BlockSpec((BM, C_DIM), lambda g, dc, ee, xsid, outid: (xsid[g], 0)),
            pl.BlockSpec((1, dcb, C_DIM), lambda g, dc, ee, xsid, outid: (ee[g], dc, 0)),
            pl.BlockSpec((1, 1, dcb), lambda g, dc, ee, xsid, outid: (ee[g], 0, dc)),
            pl.BlockSpec((1, C_DIM, dcb), lambda g, dc, ee, xsid, outid: (ee[g], 0, dc)),
            pl.BlockSpec((1, 1, C_DIM), lambda g, dc, ee, xsid, outid: (ee[g], 0, 0)),
        ],
        out_specs=pl.BlockSpec((BM, C_DIM), lambda g, dc, ee, xsid, outid: (outid[g], 0)),
    )
    return pl.pallas_call(
        _ffn_body,
        grid_spec=grid_spec,
        out_shape=jax.ShapeDtypeStruct((XS_ROWS, C_DIM), jnp.float32),
    )(ee, xsid, outid, xs, W1.astype(jnp.bfloat16), b1r,
      W2.astype(jnp.bfloat16), b2r)


def _combine_body(osort, pos_hbm, w_hbm, out_hbm,
                  posA, posB, wA, wB, bufA, bufB, obuf, sem):
    wid = lax.axis_index("s") * 2 + lax.axis_index("c")
    lane = lax.iota(jnp.int32, 16)
    tb = wid * T_PER_W
    pltpu.sync_copy(pos_hbm.at[pl.ds(tb, T_PER_W)], posA)
    pltpu.sync_copy(pos_hbm.at[pl.ds(N_TOK + tb, T_PER_W)], posB)
    pltpu.sync_copy(w_hbm.at[pl.ds(tb, T_PER_W)], wA)
    pltpu.sync_copy(w_hbm.at[pl.ds(N_TOK + tb, T_PER_W)], wB)
    for j in range(T_PER_W // 32):
        pltpu.async_copy(osort.at[posA.at[pl.ds(j * 32, 32)]], bufA, sem).wait()
        pltpu.async_copy(osort.at[posB.at[pl.ds(j * 32, 32)]], bufB, sem).wait()

        def row_body(r, _):
            rw = j * 32 + r
            wav = wA[pl.ds((rw // 16) * 16, 16)]
            wbv = wB[pl.ds((rw // 16) * 16, 16)]
            sel = lane == (rw % 16)
            wa = jnp.broadcast_to(
                lax.reduce_sum_p.bind(jnp.where(sel, wav, 0.0), axes=(0,)), (16,)
            )
            wb = jnp.broadcast_to(
                lax.reduce_sum_p.bind(jnp.where(sel, wbv, 0.0), axes=(0,)), (16,)
            )

            def col_body(cc, _c):
                a = bufA[r, pl.ds(cc * 16, 16)]
                b = bufB[r, pl.ds(cc * 16, 16)]
                obuf[r, pl.ds(cc * 16, 16)] = wa * a + wb * b
                return 0

            lax.fori_loop(0, C_DIM // 16, col_body, 0)
            return 0

        lax.fori_loop(0, 32, row_body, 0)
        pltpu.sync_copy(obuf, out_hbm.at[pl.ds(tb + j * 32, 32)])


def _combine(osort, pos, wflat):
    mesh = plsc.VectorSubcoreMesh(core_axis_name="c", subcore_axis_name="s")
    f = functools.partial(
        pl.kernel,
        out_type=jax.ShapeDtypeStruct((N_TOK, C_DIM), jnp.float32),
        mesh=mesh,
        compiler_params=pltpu.CompilerParams(needs_layout_passes=False),
        scratch_types=[
            pltpu.VMEM((T_PER_W,), jnp.int32),
            pltpu.VMEM((T_PER_W,), jnp.int32),
            pltpu.VMEM((T_PER_W,), jnp.float32),
            pltpu.VMEM((T_PER_W,), jnp.float32),
            pltpu.VMEM((32, C_DIM), jnp.float32),
            pltpu.VMEM((32, C_DIM), jnp.float32),
            pltpu.VMEM((32, C_DIM), jnp.float32),
            pltpu.SemaphoreType.DMA,
        ],
    )(_combine_body)
    return f(osort, pos, wflat)


def kernel(x, Wr, W1, b1, W2, b2):
    bx, tx, cx = x.shape
    x_flat = x.reshape(bx * tx, cx)
    idxT, wT = _router(x_flat, Wr)
    eflat = idxT.reshape(2 * N_TOK)
    wflat = wT.reshape(2 * N_TOK)
    pos, xs, ee, xsid, outid = _dispatch(eflat, x_flat)
    osort = _ffn(ee, xsid, outid, xs, W1, b1, W2, b2)
    out = _combine(osort, pos, wflat)
    aux_loss = jnp.zeros((), dtype=x.dtype)
    return (out.reshape(bx, tx, cx), aux_loss)


# speedup vs baseline: 1.2508x; 1.2508x over previous
import functools
import math

import jax
import jax.numpy as jnp
from jax import lax
from jax.experimental import pallas as pl
from jax.experimental.pallas import tpu as pltpu
from jax.experimental.pallas import tpu_sc as plsc

N_TOK = 4096
C_DIM = 1024
E_NUM = 8
D_DIM = 3072
BM = 512
G_BLOCKS = 23
XS_ROWS = (G_BLOCKS + 1) * BM
NW = 32
A_PER_W = 2 * N_TOK // NW
T_PER_W = N_TOK // NW


def _router_body(x_ref, wr_ref, idxT_ref, wT_ref):
    xb = x_ref[...]
    wr = wr_ref[...]
    logits = lax.dot_general(
        xb, wr, (((1,), (1,)), ((), ())), preferred_element_type=jnp.float32
    )
    bt, e = logits.shape
    iota_e = lax.broadcasted_iota(jnp.int32, (bt, e), 1)
    i1 = jnp.argmax(logits, axis=1)
    m1 = jnp.max(logits, axis=1)
    masked = jnp.where(iota_e == i1[:, None], -jnp.inf, logits)
    i2 = jnp.argmax(masked, axis=1)
    m2 = jnp.max(masked, axis=1)
    w0 = 1.0 / (1.0 + jnp.exp(m2 - m1))
    w1 = 1.0 - w0
    idxT_ref[...] = jnp.concatenate(
        [i1.astype(jnp.int32)[None, :], i2.astype(jnp.int32)[None, :]], axis=0
    )
    wT_ref[...] = jnp.concatenate([w0[None, :], w1[None, :]], axis=0)


def _router(x_flat, Wr):
    nb = N_TOK // BM
    return pl.pallas_call(
        _router_body,
        grid=(nb,),
        in_specs=[
            pl.BlockSpec((BM, C_DIM), lambda tb: (tb, 0)),
            pl.BlockSpec((E_NUM, C_DIM), lambda tb: (0, 0)),
        ],
        out_specs=[
            pl.BlockSpec((2, BM), lambda tb: (0, tb)),
            pl.BlockSpec((2, BM), lambda tb: (0, tb)),
        ],
        out_shape=[
            jax.ShapeDtypeStruct((2, N_TOK), jnp.int32),
            jax.ShapeDtypeStruct((2, N_TOK), jnp.float32),
        ],
    )(x_flat, Wr)


def _splat(vec16, e, lane):
    s = lax.reduce_sum_p.bind(
        jnp.where(lane == e, vec16, 0), axes=(0,)
    )
    return jnp.broadcast_to(s, (16,))


def _dispatch_body(eflat, x_hbm, pos_hbm, xs_hbm, ee_hbm, xsid_hbm, outid_hbm,
                   ev_all, posflat, pos2d, xbuf, xbuf2, m_ee, m_xs, m_out,
                   semL0, semL1, semS0, semS1):
    wid = lax.axis_index("s") * 2 + lax.axis_index("c")
    lane = lax.iota(jnp.int32, 16)
    pltpu.sync_copy(eflat, ev_all)

    start_chunk = wid * (A_PER_W // 16)

    def scan_body(i, carry):
        hist, pre = carry
        pre = jnp.where(jnp.broadcast_to(4 * i == start_chunk, (16,)), hist, pre)
        for k in range(4):
            v = ev_all[pl.ds((4 * i + k) * 16, 16)]
            for e in range(E_NUM):
                pc = plsc.all_reduce_population_count(v == e)
                hist = jnp.where(lane == e, hist + pc, hist)
        return hist, pre

    zeros16 = jnp.zeros((16,), jnp.int32)
    hist, pre = lax.fori_loop(0, (2 * N_TOK) // 64, scan_body, (zeros16, zeros16))

    nb = (hist + (BM - 1)) >> 9
    nb = jnp.where(lane < E_NUM, nb, 0)
    blk_incl = plsc.cumsum(nb)
    blk_off = blk_incl - nb
    base_lane = blk_off * BM + pre

    bases = [_splat(base_lane, e, lane) for e in range(E_NUM)]
    ends = [_splat(blk_incl, e, lane) for e in range(E_NUM)]
    a_tot = ends[E_NUM - 1]

    for ci in range(2):
        bvec = lane + ci * 16
        eob = jnp.zeros((16,), jnp.int32)
        for e in range(E_NUM):
            eob = eob + jnp.where(bvec >= ends[e], 1, 0)
        act = bvec < a_tot
        m_ee[pl.ds(ci * 16, 16)] = jnp.minimum(eob, E_NUM - 1)
        m_xs[pl.ds(ci * 16, 16)] = jnp.where(act, bvec, a_tot - 1)
        m_out[pl.ds(ci * 16, 16)] = jnp.where(
            act, bvec, jnp.broadcast_to(G_BLOCKS, (16,))
        )

    @pl.when(wid == 0)
    def _write_meta():
        pltpu.sync_copy(m_ee, ee_hbm)
        pltpu.sync_copy(m_xs, xsid_hbm)
        pltpu.sync_copy(m_out, outid_hbm)

    for i in range(A_PER_W // 16):
        v = ev_all[pl.ds((start_chunk + i) * 16, 16)]
        pos = jnp.zeros((16,), jnp.int32)
        for e in range(E_NUM):
            m = v == e
            cs = plsc.cumsum(m.astype(jnp.int32))
            pos = jnp.where(m, bases[e] + cs - 1, pos)
            bases[e] = bases[e] + plsc.all_reduce_population_count(m)
        posflat[pl.ds(i * 16, 16)] = pos
        pos2d[i // 2, pl.ds((i % 2) * 16, 16)] = pos

    pltpu.sync_copy(posflat, pos_hbm.at[pl.ds(wid * A_PER_W, A_PER_W)])

    rowbase = (wid % 16) * A_PER_W
    nj = A_PER_W // 32
    xb = [xbuf, xbuf2]
    semL = [semL0, semL1]
    semS = [semS0, semS1]
    hL = [None, None]
    hS = [None, None]
    hL[0] = pltpu.async_copy(x_hbm.at[pl.ds(rowbase, 32)], xb[0], semL[0])
    for j in range(nj):
        s = j % 2
        o = 1 - s
        if j + 1 < nj:
            if hS[o] is not None:
                hS[o].wait()
            hL[o] = pltpu.async_copy(
                x_hbm.at[pl.ds(rowbase + (j + 1) * 32, 32)], xb[o], semL[o]
            )
        hL[s].wait()
        hS[s] = pltpu.async_copy(xb[s], xs_hbm.at[pos2d.at[j]], semS[s])
    hS[0].wait()
    hS[1].wait()


def _dispatch(eflat, x_flat):
    mesh = plsc.VectorSubcoreMesh(core_axis_name="c", subcore_axis_name="s")
    f = functools.partial(
        pl.kernel,
        out_type=[
            jax.ShapeDtypeStruct((2 * N_TOK,), jnp.int32),
            jax.ShapeDtypeStruct((XS_ROWS, C_DIM), jnp.float32),
            jax.ShapeDtypeStruct((32,), jnp.int32),
            jax.ShapeDtypeStruct((32,), jnp.int32),
            jax.ShapeDtypeStruct((32,), jnp.int32),
        ],
        mesh=mesh,
        compiler_params=pltpu.CompilerParams(needs_layout_passes=False),
        scratch_types=[
            pltpu.VMEM((2 * N_TOK,), jnp.int32),
            pltpu.VMEM((A_PER_W,), jnp.int32),
            pltpu.VMEM((A_PER_W // 32, 32), jnp.int32),
            pltpu.VMEM((32, C_DIM), jnp.float32),
            pltpu.VMEM((32, C_DIM), jnp.float32),
            pltpu.VMEM((32,), jnp.int32),
            pltpu.VMEM((32,), jnp.int32),
            pltpu.VMEM((32,), jnp.int32),
            pltpu.SemaphoreType.DMA,
            pltpu.SemaphoreType.DMA,
            pltpu.SemaphoreType.DMA,
            pltpu.SemaphoreType.DMA,
        ],
    )(_dispatch_body)
    return f(eflat, x_flat)


def _ffn_body(ee_ref, xsid_ref, outid_ref, xs_ref, w1_ref, b1_ref, w2_ref,
              b2_ref, out_ref):
    g = pl.program_id(0)
    dc = pl.program_id(1)
    active = outid_ref[g] != G_BLOCKS

    @pl.when(active)
    def _():
        xb = xs_ref[...]
        w1 = w1_ref[0]
        h = lax.dot_general(
            xb, w1, (((1,), (1,)), ((), ())), preferred_element_type=jnp.float32
        ) + b1_ref[0]
        h = 0.5 * h * (1.0 + lax.erf(h * (1.0 / math.sqrt(2.0))))
        w2 = w2_ref[0]
        o = lax.dot_general(
            h, w2, (((1,), (1,)), ((), ())), preferred_element_type=jnp.float32
        )

        @pl.when(dc == 0)
        def _init():
            out_ref[...] = o + b2_ref[0]

        @pl.when(dc > 0)
        def _acc():
            out_ref[...] = out_ref[...] + o


def _ffn(ee, xsid, outid, xs, W1, b1, W2, b2):
    dcb = 1024
    ndc = D_DIM // dcb
    b1r = b1[:, None, :]
    b2r = b2[:, None, :]
    def _deff(g, dc):
        return jnp.where(g % 2 == 0, dc, ndc - 1 - dc)

    grid_spec = pltpu.PrefetchScalarGridSpec(
        num_scalar_prefetch=3,
        grid=(G_BLOCKS, ndc),
        in_specs=[
            pl.BlockSpec((BM, C_DIM), lambda g, dc, ee, xsid, outid: (xsid[g], 0)),
            pl.BlockSpec((1, dcb, C_DIM),
                         lambda g, dc, ee, xsid, outid: (ee[g], _deff(g, dc), 0)),
            pl.BlockSpec((1, 1, dcb),
                         lambda g, dc, ee, xsid, outid: (ee[g], 0, _deff(g, dc))),
            pl.BlockSpec((1, C_DIM, dcb),
                         lambda g, dc, ee, xsid, outid: (ee[g], 0, _deff(g, dc))),
            pl.BlockSpec((1, 1, C_DIM), lambda g, dc, ee, xsid, outid: (ee[g], 0, 0)),
        ],
        out_specs=pl.BlockSpec((BM, C_DIM), lambda g, dc, ee, xsid, outid: (outid[g], 0)),
    )
    return pl.pallas_call(
        _ffn_body,
        grid_spec=grid_spec,
        out_shape=jax.ShapeDtypeStruct((XS_ROWS, C_DIM), jnp.float32),
    )(ee, xsid, outid, xs, W1, b1r, W2, b2r)


CH = 16


def _combine_body(osort, pos_hbm, w_hbm, out_hbm,
                  posA, posB, wA, wB, bA0, bA1, bB0, bB1,
                  sA0, sA1, sB0, sB1):
    wid = lax.axis_index("s") * 2 + lax.axis_index("c")
    tb = wid * T_PER_W
    pltpu.sync_copy(pos_hbm.at[pl.ds(tb, T_PER_W)], posA)
    pltpu.sync_copy(pos_hbm.at[pl.ds(N_TOK + tb, T_PER_W)], posB)
    pltpu.sync_copy(w_hbm.at[pl.ds(tb, T_PER_W)], wA)
    pltpu.sync_copy(w_hbm.at[pl.ds(N_TOK + tb, T_PER_W)], wB)
    nj = T_PER_W // CH
    bufA = [bA0, bA1]
    bufB = [bB0, bB1]
    semA = [sA0, sA1]
    semB = [sB0, sB1]
    hA = [None, None]
    hB = [None, None]

    def start(j):
        s = j % 2
        hA[s] = pltpu.async_copy(
            osort.at[posA.at[pl.ds(j * CH, CH)]], bufA[s], semA[s])
        hB[s] = pltpu.async_copy(
            osort.at[posB.at[pl.ds(j * CH, CH)]], bufB[s], semB[s])

    start(0)
    for j in range(nj):
        s = j % 2
        if j + 1 < nj:
            start(j + 1)
        hA[s].wait()
        hB[s].wait()
        ba, bb = bufA[s], bufB[s]

        def row_body(r, _):
            rg = j * CH + r
            idx = jnp.broadcast_to(rg, (16,))
            wa = plsc.load_gather(wA, [idx])
            wb = plsc.load_gather(wB, [idx])
            for c in range(C_DIM // 16):
                a = ba[r, pl.ds(c * 16, 16)]
                b = bb[r, pl.ds(c * 16, 16)]
                ba[r, pl.ds(c * 16, 16)] = wa * a + wb * b
            return 0

        lax.fori_loop(0, CH, row_body, 0)
        pltpu.sync_copy(ba, out_hbm.at[pl.ds(tb + j * CH, CH)])


def _combine(osort, pos, wflat):
    mesh = plsc.VectorSubcoreMesh(core_axis_name="c", subcore_axis_name="s")
    f = functools.partial(
        pl.kernel,
        out_type=jax.ShapeDtypeStruct((N_TOK, C_DIM), jnp.float32),
        mesh=mesh,
        compiler_params=pltpu.CompilerParams(needs_layout_passes=False),
        scratch_types=[
            pltpu.VMEM((T_PER_W,), jnp.int32),
            pltpu.VMEM((T_PER_W,), jnp.int32),
            pltpu.VMEM((T_PER_W,), jnp.float32),
            pltpu.VMEM((T_PER_W,), jnp.float32),
            pltpu.VMEM((CH, C_DIM), jnp.float32),
            pltpu.VMEM((CH, C_DIM), jnp.float32),
            pltpu.VMEM((CH, C_DIM), jnp.float32),
            pltpu.VMEM((CH, C_DIM), jnp.float32),
            pltpu.SemaphoreType.DMA,
            pltpu.SemaphoreType.DMA,
            pltpu.SemaphoreType.DMA,
            pltpu.SemaphoreType.DMA,
        ],
    )(_combine_body)
    return f(osort, pos, wflat)


def kernel(x, Wr, W1, b1, W2, b2):
    bx, tx, cx = x.shape
    x_flat = x.reshape(bx * tx, cx)
    idxT, wT = _router(x_flat, Wr)
    eflat = idxT.reshape(2 * N_TOK)
    wflat = wT.reshape(2 * N_TOK)
    pos, xs, ee, xsid, outid = _dispatch(eflat, x_flat)
    osort = _ffn(ee, xsid, outid, xs, W1, b1, W2, b2)
    out = _combine(osort, pos, wflat)
    aux_loss = jnp.zeros((), dtype=x.dtype)
    return (out.reshape(bx, tx, cx), aux_loss)


# speedup vs baseline: 1.3845x; 1.1069x over previous
import functools
import math

import jax
import jax.numpy as jnp
from jax import lax
from jax.experimental import pallas as pl
from jax.experimental.pallas import tpu as pltpu
from jax.experimental.pallas import tpu_sc as plsc

N_TOK = 4096
C_DIM = 1024
E_NUM = 8
D_DIM = 3072
BM = 256
BM_SHIFT = 8
G_BLOCKS = 2 * N_TOK // BM + (E_NUM - 1)
XS_ROWS = (G_BLOCKS + 1) * BM
NW = 32
A_PER_W = 2 * N_TOK // NW
T_PER_W = N_TOK // NW


def _router_body(x_ref, wr_ref, idxT_ref, wT_ref):
    xb = x_ref[...]
    wr = wr_ref[...]
    logits = lax.dot_general(
        xb, wr, (((1,), (1,)), ((), ())), preferred_element_type=jnp.float32
    )
    bt, e = logits.shape
    iota_e = lax.broadcasted_iota(jnp.int32, (bt, e), 1)
    i1 = jnp.argmax(logits, axis=1)
    m1 = jnp.max(logits, axis=1)
    masked = jnp.where(iota_e == i1[:, None], -jnp.inf, logits)
    i2 = jnp.argmax(masked, axis=1)
    m2 = jnp.max(masked, axis=1)
    w0 = 1.0 / (1.0 + jnp.exp(m2 - m1))
    w1 = 1.0 - w0
    idxT_ref[...] = jnp.concatenate(
        [i1.astype(jnp.int32)[None, :], i2.astype(jnp.int32)[None, :]], axis=0
    )
    wT_ref[...] = jnp.concatenate([w0[None, :], w1[None, :]], axis=0)


def _router(x_flat, Wr):
    nb = N_TOK // BM
    return pl.pallas_call(
        _router_body,
        grid=(nb,),
        in_specs=[
            pl.BlockSpec((BM, C_DIM), lambda tb: (tb, 0)),
            pl.BlockSpec((E_NUM, C_DIM), lambda tb: (0, 0)),
        ],
        out_specs=[
            pl.BlockSpec((2, BM), lambda tb: (0, tb)),
            pl.BlockSpec((2, BM), lambda tb: (0, tb)),
        ],
        out_shape=[
            jax.ShapeDtypeStruct((2, N_TOK), jnp.int32),
            jax.ShapeDtypeStruct((2, N_TOK), jnp.float32),
        ],
    )(x_flat, Wr)


def _splat(vec16, e, lane):
    s = lax.reduce_sum_p.bind(
        jnp.where(lane == e, vec16, 0), axes=(0,)
    )
    return jnp.broadcast_to(s, (16,))


def _dispatch_body(eflat, x_hbm, pos_hbm, xs_hbm, ee_hbm, xsid_hbm, outid_hbm,
                   ev_all, posflat, pos2d, xbuf, xbuf2, m_ee, m_xs, m_out,
                   semL0, semL1, semS0, semS1):
    wid = lax.axis_index("s") * 2 + lax.axis_index("c")
    lane = lax.iota(jnp.int32, 16)
    pltpu.sync_copy(eflat, ev_all)

    start_chunk = wid * (A_PER_W // 16)

    def scan_body(i, carry):
        hist, pre = carry
        pre = jnp.where(jnp.broadcast_to(4 * i == start_chunk, (16,)), hist, pre)
        for k in range(4):
            v = ev_all[pl.ds((4 * i + k) * 16, 16)]
            for e in range(E_NUM):
                pc = plsc.all_reduce_population_count(v == e)
                hist = jnp.where(lane == e, hist + pc, hist)
        return hist, pre

    zeros16 = jnp.zeros((16,), jnp.int32)
    hist, pre = lax.fori_loop(0, (2 * N_TOK) // 64, scan_body, (zeros16, zeros16))

    nb = (hist + (BM - 1)) >> BM_SHIFT
    nb = jnp.where(lane < E_NUM, nb, 0)
    blk_incl = plsc.cumsum(nb)
    blk_off = blk_incl - nb
    base_lane = blk_off * BM + pre

    bases = [_splat(base_lane, e, lane) for e in range(E_NUM)]
    ends = [_splat(blk_incl, e, lane) for e in range(E_NUM)]
    a_tot = ends[E_NUM - 1]

    for ci in range(3):
        bvec = lane + ci * 16
        eob = jnp.zeros((16,), jnp.int32)
        for e in range(E_NUM):
            eob = eob + jnp.where(bvec >= ends[e], 1, 0)
        act = bvec < a_tot
        m_ee[pl.ds(ci * 16, 16)] = jnp.minimum(eob, E_NUM - 1)
        m_xs[pl.ds(ci * 16, 16)] = jnp.where(act, bvec, a_tot - 1)
        m_out[pl.ds(ci * 16, 16)] = jnp.where(
            act, bvec, jnp.broadcast_to(G_BLOCKS, (16,))
        )

    @pl.when(wid == 0)
    def _write_meta():
        pltpu.sync_copy(m_ee, ee_hbm)
        pltpu.sync_copy(m_xs, xsid_hbm)
        pltpu.sync_copy(m_out, outid_hbm)

    for i in range(A_PER_W // 16):
        v = ev_all[pl.ds((start_chunk + i) * 16, 16)]
        pos = jnp.zeros((16,), jnp.int32)
        for e in range(E_NUM):
            m = v == e
            cs = plsc.cumsum(m.astype(jnp.int32))
            pos = jnp.where(m, bases[e] + cs - 1, pos)
            bases[e] = bases[e] + plsc.all_reduce_population_count(m)
        posflat[pl.ds(i * 16, 16)] = pos
        pos2d[i // 2, pl.ds((i % 2) * 16, 16)] = pos

    pltpu.sync_copy(posflat, pos_hbm.at[pl.ds(wid * A_PER_W, A_PER_W)])

    rowbase = (wid % 16) * A_PER_W
    nj = A_PER_W // 32
    xb = [xbuf, xbuf2]
    semL = [semL0, semL1]
    semS = [semS0, semS1]
    hL = [None, None]
    hS = [None, None]
    hL[0] = pltpu.async_copy(x_hbm.at[pl.ds(rowbase, 32)], xb[0], semL[0])
    for j in range(nj):
        s = j % 2
        o = 1 - s
        if j + 1 < nj:
            if hS[o] is not None:
                hS[o].wait()
            hL[o] = pltpu.async_copy(
                x_hbm.at[pl.ds(rowbase + (j + 1) * 32, 32)], xb[o], semL[o]
            )
        hL[s].wait()
        hS[s] = pltpu.async_copy(xb[s], xs_hbm.at[pos2d.at[j]], semS[s])
    hS[0].wait()
    hS[1].wait()


def _dispatch(eflat, x_flat):
    mesh = plsc.VectorSubcoreMesh(core_axis_name="c", subcore_axis_name="s")
    f = functools.partial(
        pl.kernel,
        out_type=[
            jax.ShapeDtypeStruct((2 * N_TOK,), jnp.int32),
            jax.ShapeDtypeStruct((XS_ROWS, C_DIM), jnp.float32),
            jax.ShapeDtypeStruct((48,), jnp.int32),
            jax.ShapeDtypeStruct((48,), jnp.int32),
            jax.ShapeDtypeStruct((48,), jnp.int32),
        ],
        mesh=mesh,
        compiler_params=pltpu.CompilerParams(needs_layout_passes=False),
        scratch_types=[
            pltpu.VMEM((2 * N_TOK,), jnp.int32),
            pltpu.VMEM((A_PER_W,), jnp.int32),
            pltpu.VMEM((A_PER_W // 32, 32), jnp.int32),
            pltpu.VMEM((32, C_DIM), jnp.float32),
            pltpu.VMEM((32, C_DIM), jnp.float32),
            pltpu.VMEM((48,), jnp.int32),
            pltpu.VMEM((48,), jnp.int32),
            pltpu.VMEM((48,), jnp.int32),
            pltpu.SemaphoreType.DMA,
            pltpu.SemaphoreType.DMA,
            pltpu.SemaphoreType.DMA,
            pltpu.SemaphoreType.DMA,
        ],
    )(_dispatch_body)
    return f(eflat, x_flat)


def _ffn_body(ee_ref, xsid_ref, outid_ref, xs_ref, w1_ref, b1_ref, w2_ref,
              b2_ref, out_ref):
    g = pl.program_id(0)
    active = outid_ref[g] != G_BLOCKS

    @pl.when(active)
    def _():
        xb = xs_ref[...]
        w1 = w1_ref[0]
        h = lax.dot_general(
            xb, w1, (((1,), (1,)), ((), ())), preferred_element_type=jnp.float32
        ) + b1_ref[0]
        h = 0.5 * h * (1.0 + lax.erf(h * (1.0 / math.sqrt(2.0))))
        w2 = w2_ref[0]
        out_ref[...] = lax.dot_general(
            h, w2, (((1,), (1,)), ((), ())), preferred_element_type=jnp.float32
        ) + b2_ref[0]


def _ffn(ee, xsid, outid, xs, W1, b1, W2, b2):
    b1r = b1[:, None, :]
    b2r = b2[:, None, :]
    grid_spec = pltpu.PrefetchScalarGridSpec(
        num_scalar_prefetch=3,
        grid=(G_BLOCKS,),
        in_specs=[
            pl.BlockSpec((BM, C_DIM), lambda g, ee, xsid, outid: (xsid[g], 0)),
            pl.BlockSpec((1, D_DIM, C_DIM), lambda g, ee, xsid, outid: (ee[g], 0, 0)),
            pl.BlockSpec((1, 1, D_DIM), lambda g, ee, xsid, outid: (ee[g], 0, 0)),
            pl.BlockSpec((1, C_DIM, D_DIM), lambda g, ee, xsid, outid: (ee[g], 0, 0)),
            pl.BlockSpec((1, 1, C_DIM), lambda g, ee, xsid, outid: (ee[g], 0, 0)),
        ],
        out_specs=pl.BlockSpec((BM, C_DIM), lambda g, ee, xsid, outid: (outid[g], 0)),
    )
    return pl.pallas_call(
        _ffn_body,
        grid_spec=grid_spec,
        out_shape=jax.ShapeDtypeStruct((XS_ROWS, C_DIM), jnp.float32),
    )(ee, xsid, outid, xs, W1, b1r, W2, b2r)


CH = 16


def _combine_body(osort, pos_hbm, w_hbm, out_hbm,
                  posA, posB, wA, wB, bA0, bA1, bB0, bB1,
                  sA0, sA1, sB0, sB1):
    wid = lax.axis_index("s") * 2 + lax.axis_index("c")
    tb = wid * T_PER_W
    pltpu.sync_copy(pos_hbm.at[pl.ds(tb, T_PER_W)], posA)
    pltpu.sync_copy(pos_hbm.at[pl.ds(N_TOK + tb, T_PER_W)], posB)
    pltpu.sync_copy(w_hbm.at[pl.ds(tb, T_PER_W)], wA)
    pltpu.sync_copy(w_hbm.at[pl.ds(N_TOK + tb, T_PER_W)], wB)
    nj = T_PER_W // CH
    bufA = [bA0, bA1]
    bufB = [bB0, bB1]
    semA = [sA0, sA1]
    semB = [sB0, sB1]
    hA = [None, None]
    hB = [None, None]

    def start(j):
        s = j % 2
        hA[s] = pltpu.async_copy(
            osort.at[posA.at[pl.ds(j * CH, CH)]], bufA[s], semA[s])
        hB[s] = pltpu.async_copy(
            osort.at[posB.at[pl.ds(j * CH, CH)]], bufB[s], semB[s])

    start(0)
    for j in range(nj):
        s = j % 2
        if j + 1 < nj:
            start(j + 1)
        hA[s].wait()
        hB[s].wait()
        ba, bb = bufA[s], bufB[s]

        def row_body(r, _):
            rg = j * CH + r
            idx = jnp.broadcast_to(rg, (16,))
            wa = plsc.load_gather(wA, [idx])
            wb = plsc.load_gather(wB, [idx])
            for c in range(C_DIM // 16):
                a = ba[r, pl.ds(c * 16, 16)]
                b = bb[r, pl.ds(c * 16, 16)]
                ba[r, pl.ds(c * 16, 16)] = wa * a + wb * b
            return 0

        lax.fori_loop(0, CH, row_body, 0)
        pltpu.sync_copy(ba, out_hbm.at[pl.ds(tb + j * CH, CH)])


def _combine(osort, pos, wflat):
    mesh = plsc.VectorSubcoreMesh(core_axis_name="c", subcore_axis_name="s")
    f = functools.partial(
        pl.kernel,
        out_type=jax.ShapeDtypeStruct((N_TOK, C_DIM), jnp.float32),
        mesh=mesh,
        compiler_params=pltpu.CompilerParams(needs_layout_passes=False),
        scratch_types=[
            pltpu.VMEM((T_PER_W,), jnp.int32),
            pltpu.VMEM((T_PER_W,), jnp.int32),
            pltpu.VMEM((T_PER_W,), jnp.float32),
            pltpu.VMEM((T_PER_W,), jnp.float32),
            pltpu.VMEM((CH, C_DIM), jnp.float32),
            pltpu.VMEM((CH, C_DIM), jnp.float32),
            pltpu.VMEM((CH, C_DIM), jnp.float32),
            pltpu.VMEM((CH, C_DIM), jnp.float32),
            pltpu.SemaphoreType.DMA,
            pltpu.SemaphoreType.DMA,
            pltpu.SemaphoreType.DMA,
            pltpu.SemaphoreType.DMA,
        ],
    )(_combine_body)
    return f(osort, pos, wflat)


def kernel(x, Wr, W1, b1, W2, b2):
    bx, tx, cx = x.shape
    x_flat = x.reshape(bx * tx, cx)
    idxT, wT = _router(x_flat, Wr)
    eflat = idxT.reshape(2 * N_TOK)
    wflat = wT.reshape(2 * N_TOK)
    pos, xs, ee, xsid, outid = _dispatch(eflat, x_flat)
    osort = _ffn(ee, xsid, outid, xs, W1, b1, W2, b2)
    out = _combine(osort, pos, wflat)
    aux_loss = jnp.zeros((), dtype=x.dtype)
    return (out.reshape(bx, tx, cx), aux_loss)


# speedup vs baseline: 1.4264x; 1.0303x over previous
import functools
import math

import jax
import jax.numpy as jnp
from jax import lax
from jax.experimental import pallas as pl
from jax.experimental.pallas import tpu as pltpu
from jax.experimental.pallas import tpu_sc as plsc

N_TOK = 4096
C_DIM = 1024
E_NUM = 8
D_DIM = 3072
BM = 256
BM_SHIFT = 8
G_BLOCKS = 2 * N_TOK // BM + (E_NUM - 1)
XS_ROWS = (G_BLOCKS + 1) * BM
NW = 32
A_PER_W = 2 * N_TOK // NW
T_PER_W = N_TOK // NW


def _router_body(x_ref, wr_ref, idxT_ref, wT_ref):
    xb = x_ref[...]
    wr = wr_ref[...]
    logits = lax.dot_general(
        xb, wr, (((1,), (1,)), ((), ())), preferred_element_type=jnp.float32
    )
    bt, e = logits.shape
    iota_e = lax.broadcasted_iota(jnp.int32, (bt, e), 1)
    i1 = jnp.argmax(logits, axis=1)
    m1 = jnp.max(logits, axis=1)
    masked = jnp.where(iota_e == i1[:, None], -jnp.inf, logits)
    i2 = jnp.argmax(masked, axis=1)
    m2 = jnp.max(masked, axis=1)
    w0 = 1.0 / (1.0 + jnp.exp(m2 - m1))
    w1 = 1.0 - w0
    idxT_ref[...] = jnp.concatenate(
        [i1.astype(jnp.int32)[None, :], i2.astype(jnp.int32)[None, :]], axis=0
    )
    wT_ref[...] = jnp.concatenate([w0[None, :], w1[None, :]], axis=0)


def _router(x_flat, Wr):
    nb = N_TOK // BM
    return pl.pallas_call(
        _router_body,
        grid=(nb,),
        in_specs=[
            pl.BlockSpec((BM, C_DIM), lambda tb: (tb, 0)),
            pl.BlockSpec((E_NUM, C_DIM), lambda tb: (0, 0)),
        ],
        out_specs=[
            pl.BlockSpec((2, BM), lambda tb: (0, tb)),
            pl.BlockSpec((2, BM), lambda tb: (0, tb)),
        ],
        out_shape=[
            jax.ShapeDtypeStruct((2, N_TOK), jnp.int32),
            jax.ShapeDtypeStruct((2, N_TOK), jnp.float32),
        ],
    )(x_flat, Wr)


def _splat(vec16, e, lane):
    s = lax.reduce_sum_p.bind(
        jnp.where(lane == e, vec16, 0), axes=(0,)
    )
    return jnp.broadcast_to(s, (16,))


def _dispatch_body(eflat, x_hbm, pos_hbm, xs_hbm, ee_hbm, xsid_hbm, outid_hbm,
                   ev_all, posflatA, posflatB, pos2dA, pos2dB, histr, pre1r,
                   pre2r, xbuf, xbuf2, m_ee, m_xs, m_out,
                   semL0, semL1, semSA0, semSA1, semSB0, semSB1):
    wid = lax.axis_index("s") * 2 + lax.axis_index("c")
    lane = lax.iota(jnp.int32, 16)
    pltpu.sync_copy(eflat, ev_all)

    zeros16 = jnp.zeros((16,), jnp.int32)
    ones16 = jnp.ones((16,), jnp.int32)
    histr[...] = zeros16
    s1 = wid * 2
    s2 = (N_TOK // 64) + s1

    def scan_body(i, _):
        @pl.when(i == s1)
        def _snap1():
            pre1r[...] = histr[...]

        @pl.when(i == s2)
        def _snap2():
            pre2r[...] = histr[...]

        for k in range(4):
            v = ev_all[pl.ds((4 * i + k) * 16, 16)]
            plsc.addupdate_scatter(histr, [v], ones16)
        return 0

    lax.fori_loop(0, (2 * N_TOK) // 64, scan_body, 0)
    hist = histr[...]

    nb = (hist + (BM - 1)) >> BM_SHIFT
    nb = jnp.where(lane < E_NUM, nb, 0)
    blk_incl = plsc.cumsum(nb)
    blk_off = blk_incl - nb
    base1_lane = blk_off * BM + pre1r[...]
    base2_lane = blk_off * BM + pre2r[...]

    bases1 = [_splat(base1_lane, e, lane) for e in range(E_NUM)]
    bases2 = [_splat(base2_lane, e, lane) for e in range(E_NUM)]
    ends = [_splat(blk_incl, e, lane) for e in range(E_NUM)]
    a_tot = ends[E_NUM - 1]

    for ci in range(3):
        bvec = lane + ci * 16
        eob = jnp.zeros((16,), jnp.int32)
        for e in range(E_NUM):
            eob = eob + jnp.where(bvec >= ends[e], 1, 0)
        act = bvec < a_tot
        m_ee[pl.ds(ci * 16, 16)] = jnp.minimum(eob, E_NUM - 1)
        m_xs[pl.ds(ci * 16, 16)] = jnp.where(act, bvec, a_tot - 1)
        m_out[pl.ds(ci * 16, 16)] = jnp.where(
            act, bvec, jnp.broadcast_to(G_BLOCKS, (16,))
        )

    @pl.when(wid == 0)
    def _write_meta():
        pltpu.sync_copy(m_ee, ee_hbm)
        pltpu.sync_copy(m_xs, xsid_hbm)
        pltpu.sync_copy(m_out, outid_hbm)

    for span, bases, pflat, p2d in (
        (0, bases1, posflatA, pos2dA),
        (1, bases2, posflatB, pos2dB),
    ):
        first_chunk = span * (N_TOK // 16) + wid * (T_PER_W // 16)
        for i in range(T_PER_W // 16):
            v = ev_all[pl.ds((first_chunk + i) * 16, 16)]
            pos = jnp.zeros((16,), jnp.int32)
            for e in range(E_NUM):
                m = v == e
                cs = plsc.cumsum(m.astype(jnp.int32))
                pos = jnp.where(m, bases[e] + cs - 1, pos)
                bases[e] = bases[e] + plsc.all_reduce_population_count(m)
            pflat[pl.ds(i * 16, 16)] = pos
            p2d[i // 2, pl.ds((i % 2) * 16, 16)] = pos

    pltpu.sync_copy(posflatA, pos_hbm.at[pl.ds(wid * T_PER_W, T_PER_W)])
    pltpu.sync_copy(posflatB, pos_hbm.at[pl.ds(N_TOK + wid * T_PER_W, T_PER_W)])

    rowbase = wid * T_PER_W
    nj = T_PER_W // 32
    xb = [xbuf, xbuf2]
    semL = [semL0, semL1]
    semSA = [semSA0, semSA1]
    semSB = [semSB0, semSB1]
    hL = [None, None]
    hSA = [None, None]
    hSB = [None, None]
    hL[0] = pltpu.async_copy(x_hbm.at[pl.ds(rowbase, 32)], xb[0], semL[0])
    for j in range(nj):
        s = j % 2
        o = 1 - s
        if j + 1 < nj:
            if hSA[o] is not None:
                hSA[o].wait()
                hSB[o].wait()
            hL[o] = pltpu.async_copy(
                x_hbm.at[pl.ds(rowbase + (j + 1) * 32, 32)], xb[o], semL[o]
            )
        hL[s].wait()
        hSA[s] = pltpu.async_copy(xb[s], xs_hbm.at[pos2dA.at[j]], semSA[s])
        hSB[s] = pltpu.async_copy(xb[s], xs_hbm.at[pos2dB.at[j]], semSB[s])
    for s in range(2):
        if hSA[s] is not None:
            hSA[s].wait()
            hSB[s].wait()


def _dispatch(eflat, x_flat):
    mesh = plsc.VectorSubcoreMesh(core_axis_name="c", subcore_axis_name="s")
    f = functools.partial(
        pl.kernel,
        out_type=[
            jax.ShapeDtypeStruct((2 * N_TOK,), jnp.int32),
            jax.ShapeDtypeStruct((XS_ROWS, C_DIM), jnp.float32),
            jax.ShapeDtypeStruct((48,), jnp.int32),
            jax.ShapeDtypeStruct((48,), jnp.int32),
            jax.ShapeDtypeStruct((48,), jnp.int32),
        ],
        mesh=mesh,
        compiler_params=pltpu.CompilerParams(needs_layout_passes=False),
        scratch_types=[
            pltpu.VMEM((2 * N_TOK,), jnp.int32),
            pltpu.VMEM((T_PER_W,), jnp.int32),
            pltpu.VMEM((T_PER_W,), jnp.int32),
            pltpu.VMEM((T_PER_W // 32, 32), jnp.int32),
            pltpu.VMEM((T_PER_W // 32, 32), jnp.int32),
            pltpu.VMEM((16,), jnp.int32),
            pltpu.VMEM((16,), jnp.int32),
            pltpu.VMEM((16,), jnp.int32),
            pltpu.VMEM((32, C_DIM), jnp.float32),
            pltpu.VMEM((32, C_DIM), jnp.float32),
            pltpu.VMEM((48,), jnp.int32),
            pltpu.VMEM((48,), jnp.int32),
            pltpu.VMEM((48,), jnp.int32),
            pltpu.SemaphoreType.DMA,
            pltpu.SemaphoreType.DMA,
            pltpu.SemaphoreType.DMA,
            pltpu.SemaphoreType.DMA,
            pltpu.SemaphoreType.DMA,
            pltpu.SemaphoreType.DMA,
        ],
    )(_dispatch_body)
    return f(eflat, x_flat)


def _ffn_body(ee_ref, xsid_ref, outid_ref, xs_ref, w1_ref, b1_ref, w2_ref,
              b2_ref, out_ref):
    g = pl.program_id(0)
    active = outid_ref[g] != G_BLOCKS

    @pl.when(active)
    def _():
        xb = xs_ref[...]
        w1 = w1_ref[0]
        h = lax.dot_general(
            xb, w1, (((1,), (1,)), ((), ())), preferred_element_type=jnp.float32
        ) + b1_ref[0]
        h = 0.5 * h * (1.0 + lax.erf(h * (1.0 / math.sqrt(2.0))))
        w2 = w2_ref[0]
        out_ref[...] = lax.dot_general(
            h, w2, (((1,), (1,)), ((), ())), preferred_element_type=jnp.float32
        ) + b2_ref[0]


def _ffn(ee, xsid, outid, xs, W1, b1, W2, b2):
    b1r = b1[:, None, :]
    b2r = b2[:, None, :]
    grid_spec = pltpu.PrefetchScalarGridSpec(
        num_scalar_prefetch=3,
        grid=(G_BLOCKS,),
        in_specs=[
            pl.BlockSpec((BM, C_DIM), lambda g, ee, xsid, outid: (xsid[g], 0)),
            pl.BlockSpec((1, D_DIM, C_DIM), lambda g, ee, xsid, outid: (ee[g], 0, 0)),
            pl.BlockSpec((1, 1, D_DIM), lambda g, ee, xsid, outid: (ee[g], 0, 0)),
            pl.BlockSpec((1, C_DIM, D_DIM), lambda g, ee, xsid, outid: (ee[g], 0, 0)),
            pl.BlockSpec((1, 1, C_DIM), lambda g, ee, xsid, outid: (ee[g], 0, 0)),
        ],
        out_specs=pl.BlockSpec((BM, C_DIM), lambda g, ee, xsid, outid: (outid[g], 0)),
    )
    return pl.pallas_call(
        _ffn_body,
        grid_spec=grid_spec,
        out_shape=jax.ShapeDtypeStruct((XS_ROWS, C_DIM), jnp.float32),
    )(ee, xsid, outid, xs, W1, b1r, W2, b2r)


CH = 16


def _combine_body(osort, pos_hbm, w_hbm, out_hbm,
                  posA, posB, wA, wB, bA0, bA1, bB0, bB1,
                  sA0, sA1, sB0, sB1):
    wid = lax.axis_index("s") * 2 + lax.axis_index("c")
    tb = wid * T_PER_W
    pltpu.sync_copy(pos_hbm.at[pl.ds(tb, T_PER_W)], posA)
    pltpu.sync_copy(pos_hbm.at[pl.ds(N_TOK + tb, T_PER_W)], posB)
    pltpu.sync_copy(w_hbm.at[pl.ds(tb, T_PER_W)], wA)
    pltpu.sync_copy(w_hbm.at[pl.ds(N_TOK + tb, T_PER_W)], wB)
    nj = T_PER_W // CH
    bufA = [bA0, bA1]
    bufB = [bB0, bB1]
    semA = [sA0, sA1]
    semB = [sB0, sB1]
    hA = [None, None]
    hB = [None, None]

    def start(j):
        s = j % 2
        hA[s] = pltpu.async_copy(
            osort.at[posA.at[pl.ds(j * CH, CH)]], bufA[s], semA[s])
        hB[s] = pltpu.async_copy(
            osort.at[posB.at[pl.ds(j * CH, CH)]], bufB[s], semB[s])

    start(0)
    for j in range(nj):
        s = j % 2
        if j + 1 < nj:
            start(j + 1)
        hA[s].wait()
        hB[s].wait()
        ba, bb = bufA[s], bufB[s]

        def row_body(r, _):
            rg = j * CH + r
            idx = jnp.broadcast_to(rg, (16,))
            wa = plsc.load_gather(wA, [idx])
            wb = plsc.load_gather(wB, [idx])
            for c in range(C_DIM // 16):
                a = ba[r, pl.ds(c * 16, 16)]
                b = bb[r, pl.ds(c * 16, 16)]
                ba[r, pl.ds(c * 16, 16)] = wa * a + wb * b
            return 0

        lax.fori_loop(0, CH, row_body, 0)
        pltpu.sync_copy(ba, out_hbm.at[pl.ds(tb + j * CH, CH)])


def _combine(osort, pos, wflat):
    mesh = plsc.VectorSubcoreMesh(core_axis_name="c", subcore_axis_name="s")
    f = functools.partial(
        pl.kernel,
        out_type=jax.ShapeDtypeStruct((N_TOK, C_DIM), jnp.float32),
        mesh=mesh,
        compiler_params=pltpu.CompilerParams(needs_layout_passes=False),
        scratch_types=[
            pltpu.VMEM((T_PER_W,), jnp.int32),
            pltpu.VMEM((T_PER_W,), jnp.int32),
            pltpu.VMEM((T_PER_W,), jnp.float32),
            pltpu.VMEM((T_PER_W,), jnp.float32),
            pltpu.VMEM((CH, C_DIM), jnp.float32),
            pltpu.VMEM((CH, C_DIM), jnp.float32),
            pltpu.VMEM((CH, C_DIM), jnp.float32),
            pltpu.VMEM((CH, C_DIM), jnp.float32),
            pltpu.SemaphoreType.DMA,
            pltpu.SemaphoreType.DMA,
            pltpu.SemaphoreType.DMA,
            pltpu.SemaphoreType.DMA,
        ],
    )(_combine_body)
    return f(osort, pos, wflat)


def kernel(x, Wr, W1, b1, W2, b2):
    bx, tx, cx = x.shape
    x_flat = x.reshape(bx * tx, cx)
    idxT, wT = _router(x_flat, Wr)
    eflat = idxT.reshape(2 * N_TOK)
    wflat = wT.reshape(2 * N_TOK)
    pos, xs, ee, xsid, outid = _dispatch(eflat, x_flat)
    osort = _ffn(ee, xsid, outid, xs, W1, b1, W2, b2)
    out = _combine(osort, pos, wflat)
    aux_loss = jnp.zeros((), dtype=x.dtype)
    return (out.reshape(bx, tx, cx), aux_loss)


# speedup vs baseline: 1.4281x; 1.0012x over previous
import functools
import math

import jax
import jax.numpy as jnp
from jax import lax
from jax.experimental import pallas as pl
from jax.experimental.pallas import tpu as pltpu
from jax.experimental.pallas import tpu_sc as plsc

N_TOK = 4096
C_DIM = 1024
E_NUM = 8
D_DIM = 3072
BM = 256
BM_SHIFT = 8
G_BLOCKS = 2 * N_TOK // BM + (E_NUM - 1)
XS_ROWS = (G_BLOCKS + 1) * BM
NW = 32
A_PER_W = 2 * N_TOK // NW
T_PER_W = N_TOK // NW


def _router_body(x_ref, wr_ref, idxT_ref, wT_ref):
    xb = x_ref[...]
    wr = wr_ref[...]
    logits = lax.dot_general(
        xb, wr, (((1,), (1,)), ((), ())), preferred_element_type=jnp.float32
    )
    bt, e = logits.shape
    iota_e = lax.broadcasted_iota(jnp.int32, (bt, e), 1)
    i1 = jnp.argmax(logits, axis=1)
    m1 = jnp.max(logits, axis=1)
    masked = jnp.where(iota_e == i1[:, None], -jnp.inf, logits)
    i2 = jnp.argmax(masked, axis=1)
    m2 = jnp.max(masked, axis=1)
    w0 = 1.0 / (1.0 + jnp.exp(m2 - m1))
    w1 = 1.0 - w0
    idxT_ref[...] = jnp.concatenate(
        [i1.astype(jnp.int32)[None, :], i2.astype(jnp.int32)[None, :]], axis=0
    )
    wT_ref[...] = jnp.concatenate([w0[None, :], w1[None, :]], axis=0)


def _router(x_flat, Wr):
    nb = N_TOK // BM
    return pl.pallas_call(
        _router_body,
        grid=(nb,),
        in_specs=[
            pl.BlockSpec((BM, C_DIM), lambda tb: (tb, 0)),
            pl.BlockSpec((E_NUM, C_DIM), lambda tb: (0, 0)),
        ],
        out_specs=[
            pl.BlockSpec((2, BM), lambda tb: (0, tb)),
            pl.BlockSpec((2, BM), lambda tb: (0, tb)),
        ],
        out_shape=[
            jax.ShapeDtypeStruct((2, N_TOK), jnp.int32),
            jax.ShapeDtypeStruct((2, N_TOK), jnp.float32),
        ],
    )(x_flat, Wr)


def _splat(vec16, e, lane):
    s = lax.reduce_sum_p.bind(
        jnp.where(lane == e, vec16, 0), axes=(0,)
    )
    return jnp.broadcast_to(s, (16,))


def _dispatch_body(eflat, x_hbm, pos_hbm, xs_hbm, ee_hbm, xsid_hbm, outid_hbm,
                   ev_all, posflatA, posflatB, pos2dA, pos2dB, histr, pre1r,
                   pre2r, xbuf, xbuf2, m_ee, m_xs, m_out,
                   semL0, semL1, semSA0, semSA1, semSB0, semSB1):
    wid = lax.axis_index("s") * 2 + lax.axis_index("c")
    lane = lax.iota(jnp.int32, 16)
    pltpu.sync_copy(eflat, ev_all)

    zeros16 = jnp.zeros((16,), jnp.int32)
    ones16 = jnp.ones((16,), jnp.int32)
    histr[...] = zeros16
    s1 = wid * 2
    s2 = (N_TOK // 64) + s1

    def scan_body(i, _):
        @pl.when(i == s1)
        def _snap1():
            pre1r[...] = histr[...]

        @pl.when(i == s2)
        def _snap2():
            pre2r[...] = histr[...]

        for k in range(4):
            v = ev_all[pl.ds((4 * i + k) * 16, 16)]
            plsc.addupdate_scatter(histr, [v], ones16)
        return 0

    lax.fori_loop(0, (2 * N_TOK) // 64, scan_body, 0)
    hist = histr[...]

    nb = (hist + (BM - 1)) >> BM_SHIFT
    nb = jnp.where(lane < E_NUM, nb, 0)
    blk_incl = plsc.cumsum(nb)
    blk_off = blk_incl - nb
    base1_lane = blk_off * BM + pre1r[...]
    base2_lane = blk_off * BM + pre2r[...]

    bases1 = [_splat(base1_lane, e, lane) for e in range(E_NUM)]
    bases2 = [_splat(base2_lane, e, lane) for e in range(E_NUM)]
    ends = [_splat(blk_incl, e, lane) for e in range(E_NUM)]
    a_tot = ends[E_NUM - 1]

    for ci in range(3):
        bvec = lane + ci * 16
        eob = jnp.zeros((16,), jnp.int32)
        for e in range(E_NUM):
            eob = eob + jnp.where(bvec >= ends[e], 1, 0)
        act = bvec < a_tot
        m_ee[pl.ds(ci * 16, 16)] = jnp.minimum(eob, E_NUM - 1)
        m_xs[pl.ds(ci * 16, 16)] = jnp.where(act, bvec, a_tot - 1)
        m_out[pl.ds(ci * 16, 16)] = jnp.where(
            act, bvec, jnp.broadcast_to(G_BLOCKS, (16,))
        )

    @pl.when(wid == 0)
    def _write_meta():
        pltpu.sync_copy(m_ee, ee_hbm)
        pltpu.sync_copy(m_xs, xsid_hbm)
        pltpu.sync_copy(m_out, outid_hbm)

    for span, bases, pflat, p2d in (
        (0, bases1, posflatA, pos2dA),
        (1, bases2, posflatB, pos2dB),
    ):
        first_chunk = span * (N_TOK // 16) + wid * (T_PER_W // 16)
        for i in range(T_PER_W // 16):
            v = ev_all[pl.ds((first_chunk + i) * 16, 16)]
            pos = jnp.zeros((16,), jnp.int32)
            for e in range(E_NUM):
                m = v == e
                cs = plsc.cumsum(m.astype(jnp.int32))
                pos = jnp.where(m, bases[e] + cs - 1, pos)
                bases[e] = bases[e] + plsc.all_reduce_population_count(m)
            pflat[pl.ds(i * 16, 16)] = pos
            p2d[i // 2, pl.ds((i % 2) * 16, 16)] = pos

    pltpu.sync_copy(posflatA, pos_hbm.at[pl.ds(wid * T_PER_W, T_PER_W)])
    pltpu.sync_copy(posflatB, pos_hbm.at[pl.ds(N_TOK + wid * T_PER_W, T_PER_W)])

    rowbase = wid * T_PER_W
    nj = T_PER_W // 32
    xb = [xbuf, xbuf2]
    semL = [semL0, semL1]
    semSA = [semSA0, semSA1]
    semSB = [semSB0, semSB1]
    hL = [None, None]
    hSA = [None, None]
    hSB = [None, None]
    hL[0] = pltpu.async_copy(x_hbm.at[pl.ds(rowbase, 32)], xb[0], semL[0])
    for j in range(nj):
        s = j % 2
        o = 1 - s
        if j + 1 < nj:
            if hSA[o] is not None:
                hSA[o].wait()
                hSB[o].wait()
            hL[o] = pltpu.async_copy(
                x_hbm.at[pl.ds(rowbase + (j + 1) * 32, 32)], xb[o], semL[o]
            )
        hL[s].wait()
        hSA[s] = pltpu.async_copy(xb[s], xs_hbm.at[pos2dA.at[j]], semSA[s])
        hSB[s] = pltpu.async_copy(xb[s], xs_hbm.at[pos2dB.at[j]], semSB[s])
    for s in range(2):
        if hSA[s] is not None:
            hSA[s].wait()
            hSB[s].wait()


def _dispatch(eflat, x_flat):
    mesh = plsc.VectorSubcoreMesh(core_axis_name="c", subcore_axis_name="s")
    f = functools.partial(
        pl.kernel,
        out_type=[
            jax.ShapeDtypeStruct((2 * N_TOK,), jnp.int32),
            jax.ShapeDtypeStruct((XS_ROWS, C_DIM), jnp.float32),
            jax.ShapeDtypeStruct((48,), jnp.int32),
            jax.ShapeDtypeStruct((48,), jnp.int32),
            jax.ShapeDtypeStruct((48,), jnp.int32),
        ],
        mesh=mesh,
        compiler_params=pltpu.CompilerParams(needs_layout_passes=False),
        scratch_types=[
            pltpu.VMEM((2 * N_TOK,), jnp.int32),
            pltpu.VMEM((T_PER_W,), jnp.int32),
            pltpu.VMEM((T_PER_W,), jnp.int32),
            pltpu.VMEM((T_PER_W // 32, 32), jnp.int32),
            pltpu.VMEM((T_PER_W // 32, 32), jnp.int32),
            pltpu.VMEM((16,), jnp.int32),
            pltpu.VMEM((16,), jnp.int32),
            pltpu.VMEM((16,), jnp.int32),
            pltpu.VMEM((32, C_DIM), jnp.float32),
            pltpu.VMEM((32, C_DIM), jnp.float32),
            pltpu.VMEM((48,), jnp.int32),
            pltpu.VMEM((48,), jnp.int32),
            pltpu.VMEM((48,), jnp.int32),
            pltpu.SemaphoreType.DMA,
            pltpu.SemaphoreType.DMA,
            pltpu.SemaphoreType.DMA,
            pltpu.SemaphoreType.DMA,
            pltpu.SemaphoreType.DMA,
            pltpu.SemaphoreType.DMA,
        ],
    )(_dispatch_body)
    return f(eflat, x_flat)


def _ffn_body(ee_ref, xsid_ref, outid_ref, xs_ref, w1_ref, b1_ref, w2_ref,
              b2_ref, out_ref):
    g = pl.program_id(0)
    active = outid_ref[g] != G_BLOCKS

    @pl.when(active)
    def _():
        xb = xs_ref[...]
        w1 = w1_ref[0]
        h = lax.dot_general(
            xb, w1, (((1,), (1,)), ((), ())), preferred_element_type=jnp.float32
        ) + b1_ref[0]
        h = 0.5 * h * (1.0 + lax.erf(h * (1.0 / math.sqrt(2.0))))
        w2 = w2_ref[0]
        out_ref[...] = lax.dot_general(
            h, w2, (((1,), (1,)), ((), ())), preferred_element_type=jnp.float32
        ) + b2_ref[0]


def _ffn(ee, xsid, outid, xs, W1, b1, W2, b2):
    b1r = b1[:, None, :]
    b2r = b2[:, None, :]
    grid_spec = pltpu.PrefetchScalarGridSpec(
        num_scalar_prefetch=3,
        grid=(G_BLOCKS,),
        in_specs=[
            pl.BlockSpec((BM, C_DIM), lambda g, ee, xsid, outid: (xsid[g], 0)),
            pl.BlockSpec((1, D_DIM, C_DIM), lambda g, ee, xsid, outid: (ee[g], 0, 0)),
            pl.BlockSpec((1, 1, D_DIM), lambda g, ee, xsid, outid: (ee[g], 0, 0)),
            pl.BlockSpec((1, C_DIM, D_DIM), lambda g, ee, xsid, outid: (ee[g], 0, 0)),
            pl.BlockSpec((1, 1, C_DIM), lambda g, ee, xsid, outid: (ee[g], 0, 0)),
        ],
        out_specs=pl.BlockSpec((BM, C_DIM), lambda g, ee, xsid, outid: (outid[g], 0)),
    )
    return pl.pallas_call(
        _ffn_body,
        grid_spec=grid_spec,
        out_shape=jax.ShapeDtypeStruct((XS_ROWS, C_DIM), jnp.float32),
    )(ee, xsid, outid, xs, W1, b1r, W2, b2r)


CH = 16


def _combine_body(osort, pos_hbm, w_hbm, out_hbm,
                  posA, posB, wA, wB, bA0, bA1, bB0, bB1,
                  sA0, sA1, sB0, sB1):
    wid = lax.axis_index("s") * 2 + lax.axis_index("c")
    tb = wid * T_PER_W
    h1 = pltpu.async_copy(pos_hbm.at[pl.ds(tb, T_PER_W)], posA, sA0)
    h2 = pltpu.async_copy(pos_hbm.at[pl.ds(N_TOK + tb, T_PER_W)], posB, sA1)
    h3 = pltpu.async_copy(w_hbm.at[pl.ds(tb, T_PER_W)], wA, sB0)
    h4 = pltpu.async_copy(w_hbm.at[pl.ds(N_TOK + tb, T_PER_W)], wB, sB1)
    h1.wait()
    h2.wait()
    h3.wait()
    h4.wait()
    nj = T_PER_W // CH
    bufA = [bA0, bA1]
    bufB = [bB0, bB1]
    semA = [sA0, sA1]
    semB = [sB0, sB1]
    hA = [None, None]
    hB = [None, None]

    def start(j):
        s = j % 2
        hA[s] = pltpu.async_copy(
            osort.at[posA.at[pl.ds(j * CH, CH)]], bufA[s], semA[s])
        hB[s] = pltpu.async_copy(
            osort.at[posB.at[pl.ds(j * CH, CH)]], bufB[s], semB[s])

    start(0)
    for j in range(nj):
        s = j % 2
        if j + 1 < nj:
            start(j + 1)
        hA[s].wait()
        hB[s].wait()
        ba, bb = bufA[s], bufB[s]

        def row_body(r, _):
            rg = j * CH + r
            idx = jnp.broadcast_to(rg, (16,))
            wa = plsc.load_gather(wA, [idx])
            wb = plsc.load_gather(wB, [idx])
            for c in range(C_DIM // 16):
                a = ba[r, pl.ds(c * 16, 16)]
                b = bb[r, pl.ds(c * 16, 16)]
                ba[r, pl.ds(c * 16, 16)] = wa * a + wb * b
            return 0

        lax.fori_loop(0, CH, row_body, 0)
        pltpu.sync_copy(ba, out_hbm.at[pl.ds(tb + j * CH, CH)])


def _combine(osort, pos, wflat):
    mesh = plsc.VectorSubcoreMesh(core_axis_name="c", subcore_axis_name="s")
    f = functools.partial(
        pl.kernel,
        out_type=jax.ShapeDtypeStruct((N_TOK, C_DIM), jnp.float32),
        mesh=mesh,
        compiler_params=pltpu.CompilerParams(needs_layout_passes=False),
        scratch_types=[
            pltpu.VMEM((T_PER_W,), jnp.int32),
            pltpu.VMEM((T_PER_W,), jnp.int32),
            pltpu.VMEM((T_PER_W,), jnp.float32),
            pltpu.VMEM((T_PER_W,), jnp.float32),
            pltpu.VMEM((CH, C_DIM), jnp.float32),
            pltpu.VMEM((CH, C_DIM), jnp.float32),
            pltpu.VMEM((CH, C_DIM), jnp.float32),
            pltpu.VMEM((CH, C_DIM), jnp.float32),
            pltpu.SemaphoreType.DMA,
            pltpu.SemaphoreType.DMA,
            pltpu.SemaphoreType.DMA,
            pltpu.SemaphoreType.DMA,
        ],
    )(_combine_body)
    return f(osort, pos, wflat)


def kernel(x, Wr, W1, b1, W2, b2):
    bx, tx, cx = x.shape
    x_flat = x.reshape(bx * tx, cx)
    idxT, wT = _router(x_flat, Wr)
    eflat = idxT.reshape(2 * N_TOK)
    wflat = wT.reshape(2 * N_TOK)
    pos, xs, ee, xsid, outid = _dispatch(eflat, x_flat)
    osort = _ffn(ee, xsid, outid, xs, W1, b1, W2, b2)
    out = _combine(osort, pos, wflat)
    aux_loss = jnp.zeros((), dtype=x.dtype)
    return (out.reshape(bx, tx, cx), aux_loss)
